# baseline (speedup 1.0000x reference)
import functools
import math

import jax
import jax.numpy as jnp
from jax.experimental import pallas as pl
from jax.experimental.pallas import tpu as pltpu
from jax.experimental.pallas import tpu_sc as plsc

_B, _T, _H = 4, 8192, 1024
_M = 256
_K = 128
_V = 64
_TB = 512
_INV_SQRT_H = 1.0 / math.sqrt(_H)


def _gates_body(q_ref, wf_ref, bf_ref, wr_ref, br_ref, wq_ref, bq_ref, wk_ref,
                bk_ref, fs_ref, rs_ref, evict_ref, qkt_ref, qb_ref):
    q = q_ref[...]
    fl = jax.lax.dot_general(q, wf_ref[...], (((1,), (0,)), ((), ())),
                             preferred_element_type=jnp.float32)
    fs = jax.nn.sigmoid(fl + bf_ref[...])
    fs_ref[...] = fs
    mn = jnp.min(fs, axis=1, keepdims=True)
    iota = jax.lax.broadcasted_iota(jnp.int32, (_B, _M), 1)
    evict_ref[...] = jnp.min(jnp.where(fs == mn, iota, _M), axis=1,
                             keepdims=True)
    rl = jnp.sum(q * wr_ref[...], axis=1, keepdims=True) + br_ref[...]
    rs_ref[...] = jax.nn.sigmoid(rl)
    qq = jax.lax.dot_general(q, wq_ref[...], (((1,), (0,)), ((), ())),
                             preferred_element_type=jnp.float32) + bq_ref[...]
    qkt_ref[...] = jax.lax.dot_general(wk_ref[...], qq, (((1,), (1,)), ((), ())),
                                       preferred_element_type=jnp.float32)
    qb_ref[...] = jnp.sum(qq * bk_ref[...], axis=1, keepdims=True)


def _score_body(enc_ref, ww_ref, bw_ref, ws_ref):
    x = enc_ref[0]
    l = jax.lax.dot_general(x, ww_ref[...], (((1,), (0,)), ((), ())),
                            preferred_element_type=jnp.float32)
    ws_ref[...] = jax.nn.sigmoid(l + bw_ref[0, 0])


_TR, _TC = 2 * _B, _T // 2


def _topk_body(ws_ref, idxt_ref, scr_ref, gi_ref):
    scr_ref[...] = ws_ref[...]
    row = jax.lax.broadcasted_iota(jnp.int32, (_TR, _TC), 0)
    lane = jax.lax.broadcasted_iota(jnp.int32, (_TR, _TC), 1)
    gi_ref[...] = lane + (row // _B) * _TC

    def body(i, c):
        s = scr_ref[...]
        gi = gi_ref[...]
        rm = jnp.max(s, axis=1, keepdims=True)
        m8 = jnp.maximum(rm, pltpu.roll(rm, 4, 0))
        cand = jnp.where(s == m8, gi, _T)
        cmin = jnp.min(cand, axis=1, keepdims=True)
        idx8 = jnp.minimum(cmin, pltpu.roll(cmin, 4, 0))
        idxt_ref[pl.ds(i, 1), :] = idx8[0:_B].reshape(1, _B)
        scr_ref[...] = jnp.where(gi == idx8, -1.0, s)
        return c

    jax.lax.fori_loop(0, _K, body, 0)


_NW = 32
_RPW = _B * _K // _NW


def _sc_gather_factory():
    mesh = plsc.VectorSubcoreMesh(core_axis_name="c", subcore_axis_name="s")

    @functools.partial(
        pl.kernel,
        mesh=mesh,
        out_type=jax.ShapeDtypeStruct((_B * _M, _H), jnp.float32),
        scratch_types=[
            pltpu.VMEM((_RPW,), jnp.int32),
            pltpu.VMEM((_RPW,), jnp.int32),
            pltpu.VMEM((_RPW, _H), jnp.float32),
            pltpu.VMEM((_RPW, _H), jnp.float32),
            pltpu.SemaphoreType.DMA,
        ],
    )
    def sc_gather(gidx_hbm, evidx_hbm, zeros_hbm, enc_hbm, out_hbm,
                  idx_v, evidx_v, rows_v, zeros_v, sem):
        wid = jax.lax.axis_index("s") * 2 + jax.lax.axis_index("c")
        b = wid // 8
        dst_base = b * _M + (wid % 8) * _RPW
        zero_base = b * _M + _K + (wid % 8) * _RPW
        pltpu.sync_copy(gidx_hbm.at[pl.ds(wid * _RPW, _RPW)], idx_v)
        pltpu.sync_copy(evidx_hbm.at[wid], evidx_v)
        pltpu.sync_copy(zeros_hbm, zeros_v)
        pltpu.async_copy(enc_hbm.at[idx_v], rows_v, sem).wait()
        pltpu.sync_copy(rows_v, out_hbm.at[pl.ds(dst_base, _RPW)])
        pltpu.sync_copy(zeros_v, out_hbm.at[pl.ds(zero_base, _RPW)])
        pltpu.async_copy(zeros_v, out_hbm.at[evidx_v], sem).wait()

    return sc_gather


def _read_body(mem_ref, qkt_ref, qb_ref, rs_ref, q_ref, wo_ref, bo_ref,
               out_ref):
    for b in range(_B):
        mem_b = mem_ref[b]
        qk_b = qkt_ref[:, pl.ds(b, 1)]
        raw = jax.lax.dot_general(mem_b, qk_b, (((1,), (0,)), ((), ())),
                                  preferred_element_type=jnp.float32)
        raw = (raw + qb_ref[pl.ds(b, 1), :]) * _INV_SQRT_H
        mx = jnp.max(raw, axis=0, keepdims=True)
        e = jnp.exp(raw - mx)
        attn = e / jnp.sum(e, axis=0, keepdims=True)
        retr = jax.lax.dot_general(attn, mem_b, (((0,), (0,)), ((), ())),
                                   preferred_element_type=jnp.float32)
        rs_b = rs_ref[pl.ds(b, 1), :]
        fused = rs_b * retr + (1.0 - rs_b) * q_ref[pl.ds(b, 1), :]
        logit = jax.lax.dot_general(fused, wo_ref[...], (((1,), (0,)), ((), ())),
                                    preferred_element_type=jnp.float32)
        out_ref[pl.ds(b, 1), :] = logit + bo_ref[...]


def kernel(enc_hidden, query_hidden, slot_ages, w_write, b_write, w_read,
           b_read, w_forget, b_forget, w_q, b_q, w_k, b_k, w_out, b_out):
    del slot_ages
    bw = b_write.reshape(1, 1)
    wr_row = w_read.reshape(1, _H)
    br = b_read.reshape(1, 1)
    bf = b_forget.reshape(1, _M)
    bq = b_q.reshape(1, _H)
    bk = b_k.reshape(1, _H)
    bo = b_out.reshape(1, _V)

    fs, rs, evict, qkt, qb = pl.pallas_call(
        _gates_body,
        out_shape=[
            jax.ShapeDtypeStruct((_B, _M), jnp.float32),
            jax.ShapeDtypeStruct((_B, 1), jnp.float32),
            jax.ShapeDtypeStruct((_B, 1), jnp.int32),
            jax.ShapeDtypeStruct((_H, _B), jnp.float32),
            jax.ShapeDtypeStruct((_B, 1), jnp.float32),
        ],
    )(query_hidden, w_forget, bf, wr_row, br, w_q, bq, w_k, bk)

    ws_col = pl.pallas_call(
        _score_body,
        grid=(_B, _T // _TB),
        in_specs=[
            pl.BlockSpec((1, _TB, _H), lambda b, i: (b, i, 0)),
            pl.BlockSpec((_H, 1), lambda b, i: (0, 0)),
            pl.BlockSpec(memory_space=pltpu.SMEM),
        ],
        out_specs=pl.BlockSpec((_TB, 1),
                               lambda b, i: (b * (_T // _TB) + i, 0)),
        out_shape=jax.ShapeDtypeStruct((_B * _T, 1), jnp.float32),
    )(enc_hidden, w_write, bw)
    write_scores = ws_col.reshape(_B, _T)

    idxt = pl.pallas_call(
        _topk_body,
        out_shape=jax.ShapeDtypeStruct((_K, _B), jnp.int32),
        scratch_shapes=[pltpu.VMEM((_TR, _TC), jnp.float32),
                        pltpu.VMEM((_TR, _TC), jnp.int32)],
    )(ws_col.reshape(_B, 2, _TC).transpose(1, 0, 2).reshape(_TR, _TC))
    top_idx = idxt.T

    gidx = (top_idx + jnp.arange(_B, dtype=jnp.int32)[:, None] * _T
            ).reshape(_B * _K)
    w = jnp.arange(_NW, dtype=jnp.int32)
    wb = w // 8
    j0 = (w % 8) * _RPW
    evb = evict.reshape(_B)[wb]
    owns_evict = jnp.logical_and(evb >= j0, evb < j0 + _RPW)
    ev_target = jnp.where(owns_evict, wb * _M + evb, wb * _M + _K + j0)
    evidx = jnp.broadcast_to(ev_target[:, None], (_NW, _RPW))
    zeros_rows = jnp.zeros((_RPW, _H), jnp.float32)
    enc2 = enc_hidden.reshape(_B * _T, _H)

    mem2 = _sc_gather_factory()(gidx, evidx, zeros_rows, enc2)
    memory = mem2.reshape(_B, _M, _H)

    logits = pl.pallas_call(
        _read_body,
        out_shape=jax.ShapeDtypeStruct((_B, _V), jnp.float32),
    )(memory, qkt, qb, rs, query_hidden, w_out, bo)

    return (logits, write_scores, rs, fs, memory)


# speedup vs baseline: 1.0692x; 1.0692x over previous
import functools
import math

import jax
import jax.numpy as jnp
from jax.experimental import pallas as pl
from jax.experimental.pallas import tpu as pltpu
from jax.experimental.pallas import tpu_sc as plsc

_B, _T, _H = 4, 8192, 1024
_M = 256
_K = 128
_V = 64
_TB = 512
_INV_SQRT_H = 1.0 / math.sqrt(_H)


def _score_body(enc_ref, ww_ref, bw_ref, ws_ref):
    x = enc_ref[0]
    l = jax.lax.dot_general(x, ww_ref[...], (((1,), (0,)), ((), ())),
                            preferred_element_type=jnp.float32)
    ws_ref[...] = jax.nn.sigmoid(l + bw_ref[0, 0])


def _gates_topk_body(q_ref, wf_ref, bf_ref, wr_ref, br_ref, wq_ref, bq_ref,
                     wk_ref, bk_ref, ws_ref, fs_ref, rs_ref, evict_ref,
                     qkt_ref, qb_ref, idxt_ref, scr_ref):
    q = q_ref[...]
    fl = jax.lax.dot_general(q, wf_ref[...], (((1,), (0,)), ((), ())),
                             preferred_element_type=jnp.float32)
    fs = jax.nn.sigmoid(fl + bf_ref[...])
    fs_ref[...] = fs
    mn = jnp.min(fs, axis=1, keepdims=True)
    miota = jax.lax.broadcasted_iota(jnp.int32, (_B, _M), 1)
    evict_ref[...] = jnp.min(jnp.where(fs == mn, miota, _M), axis=1,
                             keepdims=True)
    rl = jnp.sum(q * wr_ref[...], axis=1, keepdims=True) + br_ref[...]
    rs_ref[...] = jax.nn.sigmoid(rl)
    qq = jax.lax.dot_general(q, wq_ref[...], (((1,), (0,)), ((), ())),
                             preferred_element_type=jnp.float32) + bq_ref[...]
    qkt_ref[...] = jax.lax.dot_general(wk_ref[...], qq, (((1,), (1,)), ((), ())),
                                       preferred_element_type=jnp.float32)
    qb_ref[...] = jnp.sum(qq * bk_ref[...], axis=1, keepdims=True)

    scr_ref[...] = ws_ref[...]
    iota = jax.lax.broadcasted_iota(jnp.int32, (_B, _T), 1)

    def body(i, c):
        s = scr_ref[...]
        m = jnp.max(s, axis=1, keepdims=True)
        idx = jnp.min(jnp.where(s == m, iota, _T), axis=1, keepdims=True)
        idxt_ref[pl.ds(i, 1), :] = idx.reshape(1, _B)
        scr_ref[...] = jnp.where(iota == idx, -1.0, s)
        return c

    jax.lax.fori_loop(0, _K, body, 0)


_NW = 32
_RPW = _B * _K // _NW


def _sc_gather_factory():
    mesh = plsc.VectorSubcoreMesh(core_axis_name="c", subcore_axis_name="s")

    @functools.partial(
        pl.kernel,
        mesh=mesh,
        out_type=jax.ShapeDtypeStruct((_B * _M, _H), jnp.float32),
        scratch_types=[
            pltpu.VMEM((_RPW,), jnp.int32),
            pltpu.VMEM((_RPW,), jnp.int32),
            pltpu.VMEM((_RPW, _H), jnp.float32),
            pltpu.VMEM((_RPW, _H), jnp.float32),
            pltpu.SemaphoreType.DMA,
        ],
    )
    def sc_gather(gidx_hbm, evidx_hbm, zeros_hbm, enc_hbm, out_hbm,
                  idx_v, evidx_v, rows_v, zeros_v, sem):
        wid = jax.lax.axis_index("s") * 2 + jax.lax.axis_index("c")
        b = wid // 8
        dst_base = b * _M + (wid % 8) * _RPW
        zero_base = b * _M + _K + (wid % 8) * _RPW
        pltpu.sync_copy(gidx_hbm.at[pl.ds(wid * _RPW, _RPW)], idx_v)
        pltpu.sync_copy(evidx_hbm.at[wid], evidx_v)
        pltpu.sync_copy(zeros_hbm, zeros_v)
        pltpu.async_copy(enc_hbm.at[idx_v], rows_v, sem).wait()
        pltpu.sync_copy(rows_v, out_hbm.at[pl.ds(dst_base, _RPW)])
        pltpu.sync_copy(zeros_v, out_hbm.at[pl.ds(zero_base, _RPW)])
        pltpu.async_copy(zeros_v, out_hbm.at[evidx_v], sem).wait()

    return sc_gather


def _read_body(mem_ref, qkt_ref, qb_ref, rs_ref, q_ref, wo_ref, bo_ref,
               out_ref):
    for b in range(_B):
        mem_b = mem_ref[b]
        qk_b = qkt_ref[:, pl.ds(b, 1)]
        raw = jax.lax.dot_general(mem_b, qk_b, (((1,), (0,)), ((), ())),
                                  preferred_element_type=jnp.float32)
        raw = (raw + qb_ref[pl.ds(b, 1), :]) * _INV_SQRT_H
        mx = jnp.max(raw, axis=0, keepdims=True)
        e = jnp.exp(raw - mx)
        attn = e / jnp.sum(e, axis=0, keepdims=True)
        retr = jax.lax.dot_general(attn, mem_b, (((0,), (0,)), ((), ())),
                                   preferred_element_type=jnp.float32)
        rs_b = rs_ref[pl.ds(b, 1), :]
        fused = rs_b * retr + (1.0 - rs_b) * q_ref[pl.ds(b, 1), :]
        logit = jax.lax.dot_general(fused, wo_ref[...], (((1,), (0,)), ((), ())),
                                    preferred_element_type=jnp.float32)
        out_ref[pl.ds(b, 1), :] = logit + bo_ref[...]


def kernel(enc_hidden, query_hidden, slot_ages, w_write, b_write, w_read,
           b_read, w_forget, b_forget, w_q, b_q, w_k, b_k, w_out, b_out):
    del slot_ages
    bw = b_write.reshape(1, 1)
    wr_row = w_read.reshape(1, _H)
    br = b_read.reshape(1, 1)
    bf = b_forget.reshape(1, _M)
    bq = b_q.reshape(1, _H)
    bk = b_k.reshape(1, _H)
    bo = b_out.reshape(1, _V)

    ws_col = pl.pallas_call(
        _score_body,
        grid=(_B, _T // _TB),
        in_specs=[
            pl.BlockSpec((1, _TB, _H), lambda b, i: (b, i, 0)),
            pl.BlockSpec((_H, 1), lambda b, i: (0, 0)),
            pl.BlockSpec(memory_space=pltpu.SMEM),
        ],
        out_specs=pl.BlockSpec((_TB, 1),
                               lambda b, i: (b * (_T // _TB) + i, 0)),
        out_shape=jax.ShapeDtypeStruct((_B * _T, 1), jnp.float32),
    )(enc_hidden, w_write, bw)
    write_scores = ws_col.reshape(_B, _T)

    fs, rs, evict, qkt, qb, idxt = pl.pallas_call(
        _gates_topk_body,
        out_shape=[
            jax.ShapeDtypeStruct((_B, _M), jnp.float32),
            jax.ShapeDtypeStruct((_B, 1), jnp.float32),
            jax.ShapeDtypeStruct((_B, 1), jnp.int32),
            jax.ShapeDtypeStruct((_H, _B), jnp.float32),
            jax.ShapeDtypeStruct((_B, 1), jnp.float32),
            jax.ShapeDtypeStruct((_K, _B), jnp.int32),
        ],
        scratch_shapes=[pltpu.VMEM((_B, _T), jnp.float32)],
    )(query_hidden, w_forget, bf, wr_row, br, w_q, bq, w_k, bk, write_scores)
    top_idx = idxt.T

    gidx = (top_idx + jnp.arange(_B, dtype=jnp.int32)[:, None] * _T
            ).reshape(_B * _K)
    w = jnp.arange(_NW, dtype=jnp.int32)
    wb = w // 8
    j0 = (w % 8) * _RPW
    evb = evict.reshape(_B)[wb]
    owns_evict = jnp.logical_and(evb >= j0, evb < j0 + _RPW)
    ev_target = jnp.where(owns_evict, wb * _M + evb, wb * _M + _K + j0)
    evidx = jnp.broadcast_to(ev_target[:, None], (_NW, _RPW))
    zeros_rows = jnp.zeros((_RPW, _H), jnp.float32)
    enc2 = enc_hidden.reshape(_B * _T, _H)

    mem2 = _sc_gather_factory()(gidx, evidx, zeros_rows, enc2)
    memory = mem2.reshape(_B, _M, _H)

    logits = pl.pallas_call(
        _read_body,
        out_shape=jax.ShapeDtypeStruct((_B, _V), jnp.float32),
    )(memory, qkt, qb, rs, query_hidden, w_out, bo)

    return (logits, write_scores, rs, fs, memory)
